# baseline (speedup 1.0000x reference)
import functools

import jax
import jax.numpy as jnp
from jax.experimental import pallas as pl
from jax.experimental.pallas import tpu as pltpu

_N = 12
_IN = 128
_HID = 64
_HEADS = 4
_NEG = -1e30


def _leaky(x):
    return jnp.where(x >= 0, x, 0.2 * x)


def _elu(x):
    return jnp.where(x > 0, x, jnp.exp(x) - 1.0)


def _attention(h_flat, At_src, At_dst, Msel, L3, G, heads):
    D = _HID
    HD = heads * D
    L = _N * heads
    R = G * _N
    X = jnp.dot(h_flat, At_src, preferred_element_type=jnp.float32)
    t_big = jnp.dot(h_flat, At_dst, preferred_element_type=jnp.float32)
    Y = (X.reshape(G, _N, L) * Msel[None, :, :]).reshape(R, L)
    s_row = jnp.dot(L3, Y, preferred_element_type=jnp.float32)
    e3 = _leaky(s_row[:, None, :] + t_big.reshape(G, _N, L))
    e3 = jnp.where(Msel[None, :, :] > 0, _NEG, e3)
    e = e3.reshape(R, L)
    m = jnp.max(e, axis=1, keepdims=True)
    p = jnp.exp(e - m)
    if heads > 1:
        SumE = (jax.lax.broadcasted_iota(jnp.int32, (L, heads), 0) % heads ==
                jax.lax.broadcasted_iota(jnp.int32, (L, heads), 1)
                ).astype(jnp.float32)
        denom = jnp.dot(p, SumE, preferred_element_type=jnp.float32)
    else:
        denom = jnp.sum(p, axis=1, keepdims=True)
    rden = 1.0 / (denom + 1e-16)
    if heads > 1:
        E = (jax.lax.broadcasted_iota(jnp.int32, (heads, HD), 1) // D ==
             jax.lax.broadcasted_iota(jnp.int32, (heads, HD), 0)
             ).astype(jnp.float32)
        expand = lambda v: jnp.dot(v, E, preferred_element_type=jnp.float32)
    else:
        expand = lambda v: v
    h3 = h_flat.reshape(G, _N, HD)
    acc0 = jnp.zeros((G, _N, HD), jnp.float32)
    acc1 = jnp.zeros((G, _N, HD), jnp.float32)
    for i in range(_N):
        pb = expand(p[:, i * heads:(i + 1) * heads])
        term = pb.reshape(G, _N, -1) * h3[:, i:i + 1, :]
        if i % 2 == 0:
            acc0 = acc0 + term
        else:
            acc1 = acc1 + term
    return (acc0 + acc1).reshape(R, HD) * expand(rden)


def _block_kernel(x_ref, W1_ref, as1_ref, ad1_ref, b1_ref, W2_ref, as2_ref,
                  ad2_ref, b2_ref, cW1_ref, cb1_ref, cW2_ref, cb2_ref,
                  msel1_ref, msel2_ref, l3_ref,
                  out_ref, *, G):
    x = x_ref[:].reshape(G * _N, _IN)
    L3 = l3_ref[:]

    h1 = jnp.dot(x, W1_ref[:], preferred_element_type=jnp.float32)
    o1 = _attention(h1, as1_ref[:], ad1_ref[:], msel1_ref[:], L3, G, _HEADS)
    x2 = _elu(o1 + b1_ref[:][None, :])

    h2 = jnp.dot(x2, W2_ref[:], preferred_element_type=jnp.float32)
    o2 = _attention(h2, as2_ref[:], ad2_ref[:], msel2_ref[:], L3, G, 1)
    x3 = _elu(o2 + b2_ref[:][None, :])

    graph = jnp.dot(L3, x3, preferred_element_type=jnp.float32) * (1.0 / _N)

    hc = jnp.maximum(
        jnp.dot(graph, cW1_ref[:], preferred_element_type=jnp.float32)
        + cb1_ref[:][None, :], 0.0)
    logits = jnp.dot(hc, cW2_ref[:], preferred_element_type=jnp.float32) \
        + cb2_ref[:][None, :]
    out_ref[:] = logits


def kernel(node_features, W1, att_src1, att_dst1, b1, W2, att_src2, att_dst2,
           b2, cls_W1, cls_b1, cls_W2, cls_b2):
    B = node_features.shape[0]
    G = 256
    grid = (B // G,)

    def tiled(a, heads):
        av = a.reshape(heads * _HID, 1)
        col_h = jnp.arange(_N * heads)[None, :] % heads
        row_h = jnp.arange(heads * _HID)[:, None] // _HID
        return jnp.where(col_h == row_h, av, 0.0)

    as1 = tiled(att_src1, _HEADS)
    ad1 = tiled(att_dst1, _HEADS)
    as2 = tiled(att_src2, 1)
    ad2 = tiled(att_dst2, 1)

    def msel(heads):
        r = jnp.arange(_N)[:, None]
        i = jnp.arange(_N * heads)[None, :] // heads
        return (r == i).astype(jnp.float32)

    msel1 = msel(_HEADS)
    msel2 = msel(1)
    l3 = (jnp.arange(G * _N)[None, :] // _N ==
          jnp.arange(G)[:, None]).astype(jnp.float32)

    full = lambda *shape: pl.BlockSpec(shape, lambda g: (0,) * len(shape))
    return pl.pallas_call(
        functools.partial(_block_kernel, G=G),
        grid=grid,
        in_specs=[
            pl.BlockSpec((G, _N, _IN), lambda g: (g, 0, 0)),
            full(_IN, _HEADS * _HID),
            full(_HEADS * _HID, _N * _HEADS),
            full(_HEADS * _HID, _N * _HEADS),
            full(_HEADS * _HID),
            full(_HEADS * _HID, _HID),
            full(_HID, _N),
            full(_HID, _N),
            full(_HID),
            full(_HID, _HID // 2),
            full(_HID // 2),
            full(_HID // 2, 1),
            full(1),
            full(_N, _N * _HEADS),
            full(_N, _N),
            full(G, G * _N),
        ],
        out_specs=pl.BlockSpec((G, 1), lambda g: (g, 0)),
        out_shape=jax.ShapeDtypeStruct((B, 1), jnp.float32),
        compiler_params=pltpu.CompilerParams(
            dimension_semantics=("parallel",)),
    )(node_features, W1, as1, ad1, b1, W2, as2, ad2, b2,
      cls_W1, cls_b1, cls_W2, cls_b2, msel1, msel2, l3)


# speedup vs baseline: 1.0246x; 1.0246x over previous
import functools

import jax
import jax.numpy as jnp
from jax.experimental import pallas as pl
from jax.experimental.pallas import tpu as pltpu

_N = 12
_IN = 128
_HID = 64
_HEADS = 4
_NEG = -1e30


def _leaky(x):
    return jnp.where(x >= 0, x, 0.2 * x)


def _elu(x):
    return jnp.where(x > 0, x, jnp.exp(x) - 1.0)


def _attention(h_flat, At_src, At_dst, Msel, L3, G, heads):
    D = _HID
    HD = heads * D
    L = _N * heads
    R = G * _N
    X = jnp.dot(h_flat, At_src, preferred_element_type=jnp.float32)
    t_big = jnp.dot(h_flat, At_dst, preferred_element_type=jnp.float32)
    Y = (X.reshape(G, _N, L) * Msel[None, :, :]).reshape(R, L)
    s_row = jnp.dot(L3, Y, preferred_element_type=jnp.float32)
    e3 = _leaky(s_row[:, None, :] + t_big.reshape(G, _N, L))
    e3 = jnp.where(Msel[None, :, :] > 0, _NEG, e3)
    e = e3.reshape(R, L)
    m = jnp.max(e, axis=1, keepdims=True)
    p = jnp.exp(e - m)
    if heads > 1:
        SumE = (jax.lax.broadcasted_iota(jnp.int32, (L, heads), 0) % heads ==
                jax.lax.broadcasted_iota(jnp.int32, (L, heads), 1)
                ).astype(jnp.float32)
        denom = jnp.dot(p, SumE, preferred_element_type=jnp.float32)
    else:
        denom = jnp.sum(p, axis=1, keepdims=True)
    rden = 1.0 / (denom + 1e-16)
    if heads > 1:
        E = (jax.lax.broadcasted_iota(jnp.int32, (heads, HD), 1) // D ==
             jax.lax.broadcasted_iota(jnp.int32, (heads, HD), 0)
             ).astype(jnp.float32)
        expand = lambda v: jnp.dot(v, E, preferred_element_type=jnp.float32)
    else:
        expand = lambda v: v
    h3 = h_flat.reshape(G, _N, HD)
    out = jnp.zeros((G, _N, HD), jnp.float32)
    for i in range(_N):
        pb = expand(p[:, i * heads:(i + 1) * heads])
        out = out + pb.reshape(G, _N, -1) * h3[:, i:i + 1, :]
    return out.reshape(R, HD) * expand(rden)


def _block_kernel(x_ref, W1_ref, as1_ref, ad1_ref, b1_ref, W2_ref, as2_ref,
                  ad2_ref, b2_ref, cW1_ref, cb1_ref, cW2_ref, cb2_ref,
                  msel1_ref, msel2_ref, l3_ref,
                  out_ref, *, G):
    x = x_ref[:].reshape(G * _N, _IN)
    L3 = l3_ref[:]

    h1 = jnp.dot(x, W1_ref[:], preferred_element_type=jnp.float32)
    o1 = _attention(h1, as1_ref[:], ad1_ref[:], msel1_ref[:], L3, G, _HEADS)
    x2 = _elu(o1 + b1_ref[:][None, :])

    h2 = jnp.dot(x2, W2_ref[:], preferred_element_type=jnp.float32)
    o2 = _attention(h2, as2_ref[:], ad2_ref[:], msel2_ref[:], L3, G, 1)
    x3 = _elu(o2 + b2_ref[:][None, :])

    graph = jnp.dot(L3, x3, preferred_element_type=jnp.float32) * (1.0 / _N)

    hc = jnp.maximum(
        jnp.dot(graph, cW1_ref[:], preferred_element_type=jnp.float32)
        + cb1_ref[:][None, :], 0.0)
    logits = jnp.dot(hc, cW2_ref[:], preferred_element_type=jnp.float32) \
        + cb2_ref[:][None, :]
    out_ref[:] = logits


def kernel(node_features, W1, att_src1, att_dst1, b1, W2, att_src2, att_dst2,
           b2, cls_W1, cls_b1, cls_W2, cls_b2):
    B = node_features.shape[0]
    G = 256
    grid = (B // G,)

    def tiled(a, heads):
        av = a.reshape(heads * _HID, 1)
        col_h = jnp.arange(_N * heads)[None, :] % heads
        row_h = jnp.arange(heads * _HID)[:, None] // _HID
        return jnp.where(col_h == row_h, av, 0.0)

    as1 = tiled(att_src1, _HEADS)
    ad1 = tiled(att_dst1, _HEADS)
    as2 = tiled(att_src2, 1)
    ad2 = tiled(att_dst2, 1)

    def msel(heads):
        r = jnp.arange(_N)[:, None]
        i = jnp.arange(_N * heads)[None, :] // heads
        return (r == i).astype(jnp.float32)

    msel1 = msel(_HEADS)
    msel2 = msel(1)
    l3 = (jnp.arange(G * _N)[None, :] // _N ==
          jnp.arange(G)[:, None]).astype(jnp.float32)

    full = lambda *shape: pl.BlockSpec(shape, lambda g: (0,) * len(shape))
    return pl.pallas_call(
        functools.partial(_block_kernel, G=G),
        grid=grid,
        in_specs=[
            pl.BlockSpec((G, _N, _IN), lambda g: (g, 0, 0)),
            full(_IN, _HEADS * _HID),
            full(_HEADS * _HID, _N * _HEADS),
            full(_HEADS * _HID, _N * _HEADS),
            full(_HEADS * _HID),
            full(_HEADS * _HID, _HID),
            full(_HID, _N),
            full(_HID, _N),
            full(_HID),
            full(_HID, _HID // 2),
            full(_HID // 2),
            full(_HID // 2, 1),
            full(1),
            full(_N, _N * _HEADS),
            full(_N, _N),
            full(G, G * _N),
        ],
        out_specs=pl.BlockSpec((G, 1), lambda g: (g, 0)),
        out_shape=jax.ShapeDtypeStruct((B, 1), jnp.float32),
        compiler_params=pltpu.CompilerParams(
            dimension_semantics=("parallel",)),
    )(node_features, W1, as1, ad1, b1, W2, as2, ad2, b2,
      cls_W1, cls_b1, cls_W2, cls_b2, msel1, msel2, l3)


# speedup vs baseline: 1.8249x; 1.7811x over previous
import functools

import jax
import jax.numpy as jnp
from jax.experimental import pallas as pl
from jax.experimental.pallas import tpu as pltpu

_N = 12
_IN = 128
_HID = 64
_HEADS = 4
_NEG = -1e30


def _leaky(x):
    return jnp.where(x >= 0, x, 0.2 * x)


def _elu(x):
    return jnp.where(x > 0, x, jnp.exp(x) - 1.0)


_BG = 8
_RG = _BG * _N


def _attention(h_flat, At_src, At_dst, Msel, L3, SpreadT, BlockM, G, heads):
    D = _HID
    L = _N * heads
    R = G * _N
    X = jnp.dot(h_flat, At_src, preferred_element_type=jnp.float32)
    t_big = jnp.dot(h_flat, At_dst, preferred_element_type=jnp.float32)
    Y = (X.reshape(G, _N, L) * Msel[None, :, :]).reshape(R, L)
    s_row = jnp.dot(L3, Y, preferred_element_type=jnp.float32)
    e3 = _leaky(s_row[:, None, :] + t_big.reshape(G, _N, L))
    e3 = jnp.where(Msel[None, :, :] > 0, _NEG, e3)
    e = e3.reshape(R, L)
    m = jnp.max(e, axis=1, keepdims=True)
    p = jnp.exp(e - m)
    if heads > 1:
        SumE = (jax.lax.broadcasted_iota(jnp.int32, (L, heads), 0) // _N ==
                jax.lax.broadcasted_iota(jnp.int32, (L, heads), 1)
                ).astype(jnp.float32)
        denom = jnp.dot(p, SumE, preferred_element_type=jnp.float32)
        TileE = (jax.lax.broadcasted_iota(jnp.int32, (heads, L), 1) // _N ==
                 jax.lax.broadcasted_iota(jnp.int32, (heads, L), 0)
                 ).astype(jnp.float32)
        rden_big = jnp.dot(1.0 / (denom + 1e-16), TileE,
                           preferred_element_type=jnp.float32)
    else:
        denom = jnp.sum(p, axis=1, keepdims=True)
        rden_big = 1.0 / (denom + 1e-16)
    p = p * rden_big
    n_g = R // _RG
    heads_out = []
    for hd in range(heads):
        ph = p[:, hd * _N:(hd + 1) * _N]
        hh = h_flat[:, hd * D:(hd + 1) * D]
        pieces = []
        for g in range(n_g):
            r0 = g * _RG
            T = jnp.dot(ph[r0:r0 + _RG], SpreadT,
                        preferred_element_type=jnp.float32)
            pieces.append(jnp.dot(T * BlockM, hh[r0:r0 + _RG],
                                  preferred_element_type=jnp.float32))
        heads_out.append(jnp.concatenate(pieces, axis=0))
    return jnp.concatenate(heads_out, axis=1)


def _block_kernel(x_ref, W1_ref, as1_ref, ad1_ref, b1_ref, W2_ref, as2_ref,
                  ad2_ref, b2_ref, cW1_ref, cb1_ref, cW2_ref, cb2_ref,
                  msel1_ref, msel2_ref, l3_ref, spread_ref, blockm_ref,
                  out_ref, *, G):
    x = x_ref[:].reshape(G * _N, _IN)
    L3 = l3_ref[:]
    SpreadT = spread_ref[:]
    BlockM = blockm_ref[:]

    h1 = jnp.dot(x, W1_ref[:], preferred_element_type=jnp.float32)
    o1 = _attention(h1, as1_ref[:], ad1_ref[:], msel1_ref[:], L3,
                    SpreadT, BlockM, G, _HEADS)
    x2 = _elu(o1 + b1_ref[:][None, :])

    h2 = jnp.dot(x2, W2_ref[:], preferred_element_type=jnp.float32)
    o2 = _attention(h2, as2_ref[:], ad2_ref[:], msel2_ref[:], L3,
                    SpreadT, BlockM, G, 1)
    x3 = _elu(o2 + b2_ref[:][None, :])

    graph = jnp.dot(L3, x3, preferred_element_type=jnp.float32) * (1.0 / _N)

    hc = jnp.maximum(
        jnp.dot(graph, cW1_ref[:], preferred_element_type=jnp.float32)
        + cb1_ref[:][None, :], 0.0)
    logits = jnp.dot(hc, cW2_ref[:], preferred_element_type=jnp.float32) \
        + cb2_ref[:][None, :]
    out_ref[:] = logits


def kernel(node_features, W1, att_src1, att_dst1, b1, W2, att_src2, att_dst2,
           b2, cls_W1, cls_b1, cls_W2, cls_b2):
    B = node_features.shape[0]
    G = 256
    grid = (B // G,)

    def tiled(a, heads):
        av = a.reshape(heads * _HID, 1)
        col_h = jnp.arange(_N * heads)[None, :] // _N
        row_h = jnp.arange(heads * _HID)[:, None] // _HID
        return jnp.where(col_h == row_h, av, 0.0)

    as1 = tiled(att_src1, _HEADS)
    ad1 = tiled(att_dst1, _HEADS)
    as2 = tiled(att_src2, 1)
    ad2 = tiled(att_dst2, 1)

    def msel(heads):
        r = jnp.arange(_N)[:, None]
        i = jnp.arange(_N * heads)[None, :] % _N
        return (r == i).astype(jnp.float32)

    msel1 = msel(_HEADS)
    msel2 = msel(1)
    l3 = (jnp.arange(G * _N)[None, :] // _N ==
          jnp.arange(G)[:, None]).astype(jnp.float32)
    spread = (jnp.arange(_RG)[None, :] % _N ==
              jnp.arange(_N)[:, None]).astype(jnp.float32)
    blockm = (jnp.arange(_RG)[:, None] // _N ==
              jnp.arange(_RG)[None, :] // _N).astype(jnp.float32)

    full = lambda *shape: pl.BlockSpec(shape, lambda g: (0,) * len(shape))
    return pl.pallas_call(
        functools.partial(_block_kernel, G=G),
        grid=grid,
        in_specs=[
            pl.BlockSpec((G, _N, _IN), lambda g: (g, 0, 0)),
            full(_IN, _HEADS * _HID),
            full(_HEADS * _HID, _N * _HEADS),
            full(_HEADS * _HID, _N * _HEADS),
            full(_HEADS * _HID),
            full(_HEADS * _HID, _HID),
            full(_HID, _N),
            full(_HID, _N),
            full(_HID),
            full(_HID, _HID // 2),
            full(_HID // 2),
            full(_HID // 2, 1),
            full(1),
            full(_N, _N * _HEADS),
            full(_N, _N),
            full(G, G * _N),
            full(_N, _RG),
            full(_RG, _RG),
        ],
        out_specs=pl.BlockSpec((G, 1), lambda g: (g, 0)),
        out_shape=jax.ShapeDtypeStruct((B, 1), jnp.float32),
        compiler_params=pltpu.CompilerParams(
            dimension_semantics=("parallel",)),
    )(node_features, W1, as1, ad1, b1, W2, as2, ad2, b2,
      cls_W1, cls_b1, cls_W2, cls_b2, msel1, msel2, l3, spread, blockm)


# speedup vs baseline: 2.0010x; 1.0965x over previous
import functools

import jax
import jax.numpy as jnp
from jax.experimental import pallas as pl
from jax.experimental.pallas import tpu as pltpu

_N = 12
_IN = 128
_HID = 64
_HEADS = 4
_NEG = -1e30


def _leaky(x):
    return jnp.where(x >= 0, x, 0.2 * x)


def _elu(x):
    return jnp.where(x > 0, x, jnp.exp(x) - 1.0)


_BG = 16
_RG = _BG * _N


def _attention(h_flat, At_src, At_dst, Msel, L3, SpreadT, BlockM, G, heads):
    D = _HID
    L = _N * heads
    R = G * _N
    X = jnp.dot(h_flat, At_src, preferred_element_type=jnp.float32)
    t_big = jnp.dot(h_flat, At_dst, preferred_element_type=jnp.float32)
    Y = (X.reshape(G, _N, L) * Msel[None, :, :]).reshape(R, L)
    s_row = jnp.dot(L3, Y, preferred_element_type=jnp.float32)
    e3 = _leaky(s_row[:, None, :] + t_big.reshape(G, _N, L))
    e3 = jnp.where(Msel[None, :, :] > 0, _NEG, e3)
    e = e3.reshape(R, L)
    m = jnp.max(e, axis=1, keepdims=True)
    p = jnp.exp(e - m)
    if heads > 1:
        SumE = (jax.lax.broadcasted_iota(jnp.int32, (L, heads), 0) // _N ==
                jax.lax.broadcasted_iota(jnp.int32, (L, heads), 1)
                ).astype(jnp.float32)
        denom = jnp.dot(p, SumE, preferred_element_type=jnp.float32)
        TileE = (jax.lax.broadcasted_iota(jnp.int32, (heads, L), 1) // _N ==
                 jax.lax.broadcasted_iota(jnp.int32, (heads, L), 0)
                 ).astype(jnp.float32)
        rden_big = jnp.dot(1.0 / (denom + 1e-16), TileE,
                           preferred_element_type=jnp.float32)
    else:
        denom = jnp.sum(p, axis=1, keepdims=True)
        rden_big = 1.0 / (denom + 1e-16)
    p = p * rden_big
    n_g = R // _RG
    heads_out = []
    for hd in range(heads):
        ph = p[:, hd * _N:(hd + 1) * _N]
        hh = h_flat[:, hd * D:(hd + 1) * D]
        pieces = []
        for g in range(n_g):
            r0 = g * _RG
            T = jnp.dot(ph[r0:r0 + _RG], SpreadT,
                        preferred_element_type=jnp.float32)
            pieces.append(jnp.dot(T * BlockM, hh[r0:r0 + _RG],
                                  preferred_element_type=jnp.float32))
        heads_out.append(jnp.concatenate(pieces, axis=0))
    return jnp.concatenate(heads_out, axis=1)


def _block_kernel(x_ref, W1_ref, as1_ref, ad1_ref, b1_ref, W2_ref, as2_ref,
                  ad2_ref, b2_ref, cW1_ref, cb1_ref, cW2_ref, cb2_ref,
                  msel1_ref, msel2_ref, l3_ref, spread_ref, blockm_ref,
                  out_ref, *, G):
    x = x_ref[:].reshape(G * _N, _IN)
    L3 = l3_ref[:]
    SpreadT = spread_ref[:]
    BlockM = blockm_ref[:]

    h1 = jnp.dot(x, W1_ref[:], preferred_element_type=jnp.float32)
    o1 = _attention(h1, as1_ref[:], ad1_ref[:], msel1_ref[:], L3,
                    SpreadT, BlockM, G, _HEADS)
    x2 = _elu(o1 + b1_ref[:][None, :])

    h2 = jnp.dot(x2, W2_ref[:], preferred_element_type=jnp.float32)
    o2 = _attention(h2, as2_ref[:], ad2_ref[:], msel2_ref[:], L3,
                    SpreadT, BlockM, G, 1)
    x3 = _elu(o2 + b2_ref[:][None, :])

    graph = jnp.dot(L3, x3, preferred_element_type=jnp.float32) * (1.0 / _N)

    hc = jnp.maximum(
        jnp.dot(graph, cW1_ref[:], preferred_element_type=jnp.float32)
        + cb1_ref[:][None, :], 0.0)
    logits = jnp.dot(hc, cW2_ref[:], preferred_element_type=jnp.float32) \
        + cb2_ref[:][None, :]
    out_ref[:] = logits


def kernel(node_features, W1, att_src1, att_dst1, b1, W2, att_src2, att_dst2,
           b2, cls_W1, cls_b1, cls_W2, cls_b2):
    B = node_features.shape[0]
    G = 256
    grid = (B // G,)

    def tiled(a, heads):
        av = a.reshape(heads * _HID, 1)
        col_h = jnp.arange(_N * heads)[None, :] // _N
        row_h = jnp.arange(heads * _HID)[:, None] // _HID
        return jnp.where(col_h == row_h, av, 0.0)

    as1 = tiled(att_src1, _HEADS)
    ad1 = tiled(att_dst1, _HEADS)
    as2 = tiled(att_src2, 1)
    ad2 = tiled(att_dst2, 1)

    def msel(heads):
        r = jnp.arange(_N)[:, None]
        i = jnp.arange(_N * heads)[None, :] % _N
        return (r == i).astype(jnp.float32)

    msel1 = msel(_HEADS)
    msel2 = msel(1)
    l3 = (jnp.arange(G * _N)[None, :] // _N ==
          jnp.arange(G)[:, None]).astype(jnp.float32)
    spread = (jnp.arange(_RG)[None, :] % _N ==
              jnp.arange(_N)[:, None]).astype(jnp.float32)
    blockm = (jnp.arange(_RG)[:, None] // _N ==
              jnp.arange(_RG)[None, :] // _N).astype(jnp.float32)

    full = lambda *shape: pl.BlockSpec(shape, lambda g: (0,) * len(shape))
    return pl.pallas_call(
        functools.partial(_block_kernel, G=G),
        grid=grid,
        in_specs=[
            pl.BlockSpec((G, _N, _IN), lambda g: (g, 0, 0)),
            full(_IN, _HEADS * _HID),
            full(_HEADS * _HID, _N * _HEADS),
            full(_HEADS * _HID, _N * _HEADS),
            full(_HEADS * _HID),
            full(_HEADS * _HID, _HID),
            full(_HID, _N),
            full(_HID, _N),
            full(_HID),
            full(_HID, _HID // 2),
            full(_HID // 2),
            full(_HID // 2, 1),
            full(1),
            full(_N, _N * _HEADS),
            full(_N, _N),
            full(G, G * _N),
            full(_N, _RG),
            full(_RG, _RG),
        ],
        out_specs=pl.BlockSpec((G, 1), lambda g: (g, 0)),
        out_shape=jax.ShapeDtypeStruct((B, 1), jnp.float32),
        compiler_params=pltpu.CompilerParams(
            dimension_semantics=("parallel",)),
    )(node_features, W1, as1, ad1, b1, W2, as2, ad2, b2,
      cls_W1, cls_b1, cls_W2, cls_b2, msel1, msel2, l3, spread, blockm)
